# Initial kernel scaffold; baseline (speedup 1.0000x reference)
import jax
import jax.numpy as jnp
from jax.experimental import pallas as pl


def kernel(bkg, bkg_mask, defect, defect_mask, defect_location_masks):
    raise NotImplementedError("write your pallas kernel here")


# speedup vs baseline: 26.5502x; 26.5502x over previous
import functools

import jax
import jax.numpy as jnp
from jax import lax
from jax.experimental import pallas as pl
from jax.experimental.pallas import tpu as pltpu

H = 512
W = 512
PAD = 256


def _shift2d(src, ty, tx, pad_ref):
    del pad_ref
    i = lax.broadcasted_iota(jnp.int32, (H, W), 0)
    j = lax.broadcasted_iota(jnp.int32, (H, W), 1)
    txp = jnp.where(tx < 0, tx + W, tx)
    typ = jnp.where(ty < 0, ty + H, ty)
    rolled = pltpu.roll(pltpu.roll(src, txp, axis=1), typ, axis=0)
    mask = (j >= tx) & (j < W + tx) & (i >= ty) & (i < H + ty)
    return jnp.where(mask, rolled, 0.0)


def _fused_kernel(bkg_ref, bkgm_ref, defect_ref, dmask_ref, loc_ref,
                  img_ref, outm_ref, pad_ref, dm_ref, sh_ref):
    c = pl.program_id(1)
    b = pl.program_id(0)

    @pl.when(jnp.logical_and(b == 0, c == 0))
    def _zero_pads():
        pad_ref[pl.ds(0, PAD), :] = jnp.zeros((PAD, W), jnp.float32)
        pad_ref[pl.ds(PAD + H, PAD), :] = jnp.zeros((PAD, W), jnp.float32)

    @pl.when(c == 0)
    def _compute_shift():
        m = loc_ref[0, 0]
        bmask = m >= jnp.float32(1e-5)
        bf = bmask.astype(jnp.float32)
        i = lax.broadcasted_iota(jnp.int32, (H, W), 0)
        j = lax.broadcasted_iota(jnp.int32, (H, W), 1)
        upper = (i <= j).astype(jnp.float32)
        rowcum = lax.dot(bf, upper, preferred_element_type=jnp.float32, precision=lax.Precision.HIGHEST)
        rtot = rowcum[:, W - 1:W]
        lstrict = (j < i).astype(jnp.float32)
        cexc = lax.dot(lstrict, rtot, preferred_element_type=jnp.float32, precision=lax.Precision.HIGHEST)
        counts = cexc + rowcum
        n = counts[H - 1, W - 1].astype(jnp.int32)
        target = (n // 2 + 1).astype(jnp.float32)
        flatidx = i * W + j
        hit = jnp.logical_and(bmask, counts == target)
        cand = jnp.where(hit, flatidx, jnp.int32(2 ** 30))
        fidx = jnp.min(cand)
        fidx = jnp.where(n == 0, 0, fidx)
        row = fidx // W
        col = fidx - row * W
        sh_ref[0] = row - H // 2
        sh_ref[1] = col - W // 2
        dm_ref[...] = _shift2d(dmask_ref[0, 0], sh_ref[0], sh_ref[1], pad_ref)

    ty = sh_ref[0]
    tx = sh_ref[1]

    @pl.when(c < 3)
    def _composite():
        d_s = _shift2d(defect_ref[0, 0], ty, tx, pad_ref)
        dm = dm_ref[...]
        img_ref[0, 0] = bkg_ref[0, 0] * (1.0 - dm) + d_s * dm

    @pl.when(c == 3)
    def _mask_out():
        outm_ref[0, 0] = jnp.clip(dm_ref[...] + bkgm_ref[0, 0], 0.0, 1.0)


@jax.jit
def kernel(bkg, bkg_mask, defect, defect_mask, defect_location_masks):
    B = bkg.shape[0]

    def ch_map(b, c):
        return (b, jnp.minimum(c, 2), 0, 0)

    def s_map(b, c):
        return (b, 0, 0, 0)

    blk = (1, 1, H, W)
    img_aug, out_mask = pl.pallas_call(
        _fused_kernel,
        grid=(B, 4),
        in_specs=[
            pl.BlockSpec(blk, ch_map),
            pl.BlockSpec(blk, s_map),
            pl.BlockSpec(blk, ch_map),
            pl.BlockSpec(blk, s_map),
            pl.BlockSpec(blk, s_map),
        ],
        out_specs=[
            pl.BlockSpec(blk, ch_map),
            pl.BlockSpec(blk, s_map),
        ],
        out_shape=[
            jax.ShapeDtypeStruct((B, 3, H, W), jnp.float32),
            jax.ShapeDtypeStruct((B, 1, H, W), jnp.float32),
        ],
        scratch_shapes=[
            pltpu.VMEM((H + 2 * PAD, W), jnp.float32),
            pltpu.VMEM((H, W), jnp.float32),
            pltpu.SMEM((2,), jnp.int32),
        ],
    )(bkg, bkg_mask, defect, defect_mask, defect_location_masks)
    return img_aug, out_mask


# speedup vs baseline: 48.2549x; 1.8175x over previous
import jax
import jax.numpy as jnp
from jax import lax
from jax.experimental import pallas as pl
from jax.experimental.pallas import tpu as pltpu

H = 512
W = 512


def _roll2d(src, typ, txp):
    return pltpu.roll(pltpu.roll(src, txp, axis=1), typ, axis=0)


def _fused_kernel(bkg_ref, bkgm_ref, defect_ref, dmask_ref, loc_ref,
                  img_ref, outm_ref):
    i = lax.broadcasted_iota(jnp.int32, (H, W), 0)
    j = lax.broadcasted_iota(jnp.int32, (H, W), 1)
    big = jnp.int32(2 ** 30)

    m = loc_ref[0, 0]
    bmask = m >= jnp.float32(1e-5)
    bf = bmask.astype(jnp.float32)
    rt = jnp.sum(bf, axis=1, keepdims=True)
    lstrict = (j < i).astype(jnp.float32)
    cexc = lax.dot(lstrict, rt, preferred_element_type=jnp.float32,
                   precision=lax.Precision.HIGHEST)
    n = (cexc[H - 1, 0] + rt[H - 1, 0]).astype(jnp.int32)
    target = (n // 2 + 1).astype(jnp.float32)
    rowhit = jnp.logical_and(cexc < target, cexc + rt >= target)
    i0 = lax.broadcasted_iota(jnp.int32, (H, 1), 0)
    istar = jnp.min(jnp.where(rowhit, i0, big))
    krow = target - jnp.sum(jnp.where(rowhit, cexc, 0.0))
    rowv = jnp.sum(jnp.where(i == istar, bf, 0.0), axis=0, keepdims=True)
    j1 = lax.broadcasted_iota(jnp.int32, (1, W), 1)
    cum = rowv
    for s in (1, 2, 4, 8, 16, 32, 64, 128, 256):
        cum = cum + jnp.where(j1 >= s, pltpu.roll(cum, s, axis=1), 0.0)
    colhit = jnp.logical_and(rowv > 0.0, cum == krow)
    cstar = jnp.min(jnp.where(colhit, j1, big))
    fidx = istar * W + cstar
    fidx = jnp.where(n == 0, 0, fidx)
    row = fidx // W
    col = fidx - row * W
    ty = row - H // 2
    tx = col - W // 2

    txp = jnp.where(tx < 0, tx + W, tx)
    typ = jnp.where(ty < 0, ty + H, ty)
    valid = (j >= tx) & (j < W + tx) & (i >= ty) & (i < H + ty)
    dm = jnp.where(valid, _roll2d(dmask_ref[0, 0], typ, txp), 0.0)
    for c in range(3):
        d_s = _roll2d(defect_ref[0, c], typ, txp)
        b = bkg_ref[0, c]
        img_ref[0, c] = b * (1.0 - dm) + d_s * dm
    outm_ref[0, 0] = jnp.clip(dm + bkgm_ref[0, 0], 0.0, 1.0)


@jax.jit
def kernel(bkg, bkg_mask, defect, defect_mask, defect_location_masks):
    B = bkg.shape[0]

    def ch_map(b):
        return (b, 0, 0, 0)

    img_aug, out_mask = pl.pallas_call(
        _fused_kernel,
        grid=(B,),
        in_specs=[
            pl.BlockSpec((1, 3, H, W), ch_map),
            pl.BlockSpec((1, 1, H, W), ch_map),
            pl.BlockSpec((1, 3, H, W), ch_map),
            pl.BlockSpec((1, 1, H, W), ch_map),
            pl.BlockSpec((1, 1, H, W), ch_map),
        ],
        out_specs=[
            pl.BlockSpec((1, 3, H, W), ch_map),
            pl.BlockSpec((1, 1, H, W), ch_map),
        ],
        out_shape=[
            jax.ShapeDtypeStruct((B, 3, H, W), jnp.float32),
            jax.ShapeDtypeStruct((B, 1, H, W), jnp.float32),
        ],
    )(bkg, bkg_mask, defect, defect_mask, defect_location_masks)
    return img_aug, out_mask
